# baseline (speedup 1.0000x reference)
import functools

import jax
import jax.numpy as jnp
from jax import lax
from jax.experimental import pallas as pl
from jax.experimental.pallas import tpu as pltpu
from jax.experimental.pallas import tpu_sc as plsc

_GLOBAL_MEAN = 3.5
_MIN_R = 1.0
_MAX_R = 5.0


def _tp_body(in_ref, eye_ref, out_ref):
    out_ref[...] = jax.lax.dot_general(
        in_ref[...], eye_ref[...], (((0,), (0,)), ((), ())),
        preferred_element_type=jnp.float32)


@jax.jit
def _tc_transpose(tab_t):
    d, u = tab_t.shape
    bn = 8192
    return pl.pallas_call(
        _tp_body,
        grid=(pl.cdiv(u, bn),),
        in_specs=[
            pl.BlockSpec((d, bn), lambda i: (0, i)),
            pl.BlockSpec((d, d), lambda i: (0, 0)),
        ],
        out_specs=pl.BlockSpec((bn, d), lambda i: (i, 0)),
        out_shape=jax.ShapeDtypeStruct((u, d), jnp.float32),
    )(tab_t, jnp.eye(d, dtype=jnp.float32))


@functools.partial(jax.jit, static_argnums=(8, 9))
def _sc_gather(uidx_hi2, iidx_hi2, uidx_hi7, iidx_hi7,
               uemb_r, iemb_r, ub_r, ib_r, B, W):
    info = plsc.get_sparse_core_info()
    nw = info.num_cores * info.num_subcores
    nc = info.num_cores
    b_per_w = B // nw
    chunk = b_per_w // 2
    mesh = plsc.VectorSubcoreMesh(core_axis_name="c", subcore_axis_name="s")

    @functools.partial(
        pl.kernel,
        out_type=(
            jax.ShapeDtypeStruct((B, W), jnp.float32),
            jax.ShapeDtypeStruct((B, W), jnp.float32),
            jax.ShapeDtypeStruct((B, W), jnp.float32),
            jax.ShapeDtypeStruct((B, W), jnp.float32),
        ),
        mesh=mesh,
        scratch_types=[
            pltpu.VMEM((b_per_w,), jnp.int32),
            pltpu.VMEM((b_per_w,), jnp.int32),
            pltpu.VMEM((b_per_w,), jnp.int32),
            pltpu.VMEM((b_per_w,), jnp.int32),
            pltpu.VMEM((chunk, W), jnp.float32),
            pltpu.VMEM((chunk, W), jnp.float32),
            pltpu.SemaphoreType.DMA,
        ],
    )
    def gather_kernel(uhi2_hbm, ihi2_hbm, uhi7_hbm, ihi7_hbm,
                      uemb_hbm, iemb_hbm, ubr_hbm, ibr_hbm,
                      urows_out, irows_out, ubr_out, ibr_out,
                      uhi2_v, ihi2_v, uhi7_v, ihi7_v,
                      rows_a, rows_b, sem):
        wid = lax.axis_index("s") * nc + lax.axis_index("c")
        base = wid * b_per_w
        bsl = pl.ds(base, b_per_w)
        pltpu.sync_copy(uhi2_hbm.at[bsl], uhi2_v)
        pltpu.sync_copy(ihi2_hbm.at[bsl], ihi2_v)
        pltpu.sync_copy(uhi7_hbm.at[bsl], uhi7_v)
        pltpu.sync_copy(ihi7_hbm.at[bsl], ihi7_v)
        for half in range(2):
            lo = half * chunk
            csl = pl.ds(lo, chunk)
            osl = pl.ds(base + lo, chunk)
            cu = pltpu.async_copy(uemb_hbm.at[uhi2_v.at[csl]], rows_a, sem)
            ci = pltpu.async_copy(iemb_hbm.at[ihi2_v.at[csl]], rows_b, sem)
            cu.wait()
            ci.wait()
            pltpu.sync_copy(rows_a, urows_out.at[osl])
            pltpu.sync_copy(rows_b, irows_out.at[osl])
        for half in range(2):
            lo = half * chunk
            csl = pl.ds(lo, chunk)
            osl = pl.ds(base + lo, chunk)
            cu = pltpu.async_copy(ubr_hbm.at[uhi7_v.at[csl]], rows_a, sem)
            ci = pltpu.async_copy(ibr_hbm.at[ihi7_v.at[csl]], rows_b, sem)
            cu.wait()
            ci.wait()
            pltpu.sync_copy(rows_a, ubr_out.at[osl])
            pltpu.sync_copy(rows_b, ibr_out.at[osl])

    return gather_kernel(uidx_hi2, iidx_hi2, uidx_hi7, iidx_hi7,
                         uemb_r, iemb_r, ub_r, ib_r)


def _mlp_body(urows_ref, irows_ref, ubr_ref, ibr_ref, us_ref, is_ref,
              ulo_ref, ilo_ref, w1u_ref, w1i_ref, b1_ref, w2_ref, b2_ref,
              w3_ref, cst_ref, out_ref):
    d = w1u_ref.shape[0]
    bm = urows_ref.shape[0]
    us = us_ref[...]
    isx = is_ref[...]
    urows = urows_ref[...]
    irows = irows_ref[...]
    uvec = jnp.zeros((bm, d), jnp.float32)
    ivec = jnp.zeros((bm, d), jnp.float32)
    for k in range(4):
        um = (us == k).astype(jnp.float32)[:, None]
        im = (isx == k).astype(jnp.float32)[:, None]
        uvec = uvec + um * urows[:, k * d:(k + 1) * d]
        ivec = ivec + im * irows[:, k * d:(k + 1) * d]
    lane = jax.lax.broadcasted_iota(jnp.int32, (bm, 128), 1)
    um = (lane == ulo_ref[...][:, None]).astype(jnp.float32)
    im = (lane == ilo_ref[...][:, None]).astype(jnp.float32)
    ub = jnp.sum(ubr_ref[...] * um, axis=1)
    ib = jnp.sum(ibr_ref[...] * im, axis=1)
    h = (jnp.dot(uvec, w1u_ref[...], preferred_element_type=jnp.float32)
         + jnp.dot(ivec, w1i_ref[...], preferred_element_type=jnp.float32)
         + b1_ref[...])
    h = jnp.maximum(h, 0.0)
    h2 = jnp.dot(h, w2_ref[...], preferred_element_type=jnp.float32) + b2_ref[...]
    h2 = jnp.maximum(h2, 0.0)
    inter = jnp.sum(h2 * w3_ref[...], axis=1)
    pred = cst_ref[0, 0] + ub + ib + inter
    out_ref[...] = jnp.clip(pred, _MIN_R, _MAX_R)


@functools.partial(jax.jit, static_argnums=(15, 16, 17, 18))
def _tc_mlp(urows, irows, ubr, ibr, us, isx, ulo, ilo,
            w1u, w1i, b1r, w2t, b2r, w3r, cst, B, D, H, W):
    bm = 2048
    grid = (B // bm,)
    return pl.pallas_call(
        _mlp_body,
        grid=grid,
        in_specs=[
            pl.BlockSpec((bm, W), lambda i: (i, 0)),
            pl.BlockSpec((bm, W), lambda i: (i, 0)),
            pl.BlockSpec((bm, W), lambda i: (i, 0)),
            pl.BlockSpec((bm, W), lambda i: (i, 0)),
            pl.BlockSpec((bm,), lambda i: (i,)),
            pl.BlockSpec((bm,), lambda i: (i,)),
            pl.BlockSpec((bm,), lambda i: (i,)),
            pl.BlockSpec((bm,), lambda i: (i,)),
            pl.BlockSpec((D, H), lambda i: (0, 0)),
            pl.BlockSpec((D, H), lambda i: (0, 0)),
            pl.BlockSpec((1, H), lambda i: (0, 0)),
            pl.BlockSpec((H, 32), lambda i: (0, 0)),
            pl.BlockSpec((1, 32), lambda i: (0, 0)),
            pl.BlockSpec((1, 32), lambda i: (0, 0)),
            pl.BlockSpec((1, 1), lambda i: (0, 0)),
        ],
        out_specs=pl.BlockSpec((bm,), lambda i: (i,)),
        out_shape=jax.ShapeDtypeStruct((B,), jnp.float32),
    )(urows, irows, ubr, ibr, us, isx, ulo, ilo,
      w1u, w1i, b1r, w2t, b2r, w3r, cst)


def kernel(user_idx, item_idx, user_emb, item_emb, user_bias, item_bias,
           W1, b1, W2, b2, W3, b3):
    B = user_idx.shape[0]
    U, D = user_emb.shape
    H = W1.shape[0]
    W = 4 * D

    uemb_r = _tc_transpose(user_emb.T).reshape(U // 4, W)
    iemb_r = _tc_transpose(item_emb.T).reshape(U // 4, W)
    uidx_hi2 = jax.lax.shift_right_logical(user_idx, 2)
    iidx_hi2 = jax.lax.shift_right_logical(item_idx, 2)
    us = jax.lax.bitwise_and(user_idx, 3)
    isx = jax.lax.bitwise_and(item_idx, 3)
    uidx_hi7 = jax.lax.shift_right_logical(user_idx, 7)
    iidx_hi7 = jax.lax.shift_right_logical(item_idx, 7)
    uidx_lo7 = jax.lax.bitwise_and(user_idx, 127)
    iidx_lo7 = jax.lax.bitwise_and(item_idx, 127)

    u_pad = (-U) % 128
    ub_r = jnp.pad(user_bias.reshape(-1), (0, u_pad)).reshape(-1, 128)
    ib_r = jnp.pad(item_bias.reshape(-1), (0, u_pad)).reshape(-1, 128)

    urows, irows, ubr, ibr = _sc_gather(
        uidx_hi2, iidx_hi2, uidx_hi7, iidx_hi7,
        uemb_r, iemb_r, ub_r, ib_r, B, W)

    w1u = W1[:, :D].T
    w1i = W1[:, D:].T
    b1r = b1.reshape(1, H)
    w2t = W2.T
    b2r = b2.reshape(1, 32)
    w3r = W3.reshape(1, 32)
    cst = (_GLOBAL_MEAN + b3).reshape(1, 1)

    return _tc_mlp(urows, irows, ubr, ibr, us, isx, uidx_lo7, iidx_lo7,
                   w1u, w1i, b1r, w2t, b2r, w3r, cst, B, D, H, W)


# speedup vs baseline: 1.1367x; 1.1367x over previous
import functools

import jax
import jax.numpy as jnp
from jax import lax
from jax.experimental import pallas as pl
from jax.experimental.pallas import tpu as pltpu
from jax.experimental.pallas import tpu_sc as plsc

_GLOBAL_MEAN = 3.5
_MIN_R = 1.0
_MAX_R = 5.0


def _tp_body(in_ref, eye_ref, out_ref):
    out_ref[...] = jax.lax.dot_general(
        in_ref[...], eye_ref[...], (((0,), (0,)), ((), ())),
        preferred_element_type=jnp.float32)


@jax.jit
def _tc_transpose(tab_t):
    d, u = tab_t.shape
    bn = 8192
    return pl.pallas_call(
        _tp_body,
        grid=(pl.cdiv(u, bn),),
        in_specs=[
            pl.BlockSpec((d, bn), lambda i: (0, i)),
            pl.BlockSpec((d, d), lambda i: (0, 0)),
        ],
        out_specs=pl.BlockSpec((bn, d), lambda i: (i, 0)),
        out_shape=jax.ShapeDtypeStruct((u, d), jnp.float32),
    )(tab_t, jnp.eye(d, dtype=jnp.float32))


@functools.partial(jax.jit, static_argnums=(8, 9))
def _sc_gather(uidx_hi2, iidx_hi2, uidx_hi7, iidx_hi7,
               uemb_r, iemb_r, ub_r, ib_r, B, W):
    info = plsc.get_sparse_core_info()
    nw = info.num_cores * info.num_subcores
    nc = info.num_cores
    b_per_w = B // nw
    chunk = b_per_w // 2
    mesh = plsc.VectorSubcoreMesh(core_axis_name="c", subcore_axis_name="s")

    @functools.partial(
        pl.kernel,
        out_type=(
            jax.ShapeDtypeStruct((B, W), jnp.float32),
            jax.ShapeDtypeStruct((B, W), jnp.float32),
            jax.ShapeDtypeStruct((B, W), jnp.float32),
            jax.ShapeDtypeStruct((B, W), jnp.float32),
        ),
        mesh=mesh,
        scratch_types=[
            pltpu.VMEM((b_per_w,), jnp.int32),
            pltpu.VMEM((b_per_w,), jnp.int32),
            pltpu.VMEM((b_per_w,), jnp.int32),
            pltpu.VMEM((b_per_w,), jnp.int32),
            pltpu.VMEM((chunk, W), jnp.float32),
            pltpu.VMEM((chunk, W), jnp.float32),
            pltpu.SemaphoreType.DMA,
        ],
    )
    def gather_kernel(uhi2_hbm, ihi2_hbm, uhi7_hbm, ihi7_hbm,
                      uemb_hbm, iemb_hbm, ubr_hbm, ibr_hbm,
                      urows_out, irows_out, ubr_out, ibr_out,
                      uhi2_v, ihi2_v, uhi7_v, ihi7_v,
                      rows_a, rows_b, sem):
        wid = lax.axis_index("s") * nc + lax.axis_index("c")
        base = wid * b_per_w
        bsl = pl.ds(base, b_per_w)
        pltpu.sync_copy(uhi2_hbm.at[bsl], uhi2_v)
        pltpu.sync_copy(ihi2_hbm.at[bsl], ihi2_v)
        pltpu.sync_copy(uhi7_hbm.at[bsl], uhi7_v)
        pltpu.sync_copy(ihi7_hbm.at[bsl], ihi7_v)
        for half in range(2):
            lo = half * chunk
            csl = pl.ds(lo, chunk)
            osl = pl.ds(base + lo, chunk)
            cu = pltpu.async_copy(uemb_hbm.at[uhi2_v.at[csl]], rows_a, sem)
            ci = pltpu.async_copy(iemb_hbm.at[ihi2_v.at[csl]], rows_b, sem)
            cu.wait()
            ci.wait()
            pltpu.sync_copy(rows_a, urows_out.at[osl])
            pltpu.sync_copy(rows_b, irows_out.at[osl])
        for half in range(2):
            lo = half * chunk
            csl = pl.ds(lo, chunk)
            osl = pl.ds(base + lo, chunk)
            cu = pltpu.async_copy(ubr_hbm.at[uhi7_v.at[csl]], rows_a, sem)
            ci = pltpu.async_copy(ibr_hbm.at[ihi7_v.at[csl]], rows_b, sem)
            cu.wait()
            ci.wait()
            pltpu.sync_copy(rows_a, ubr_out.at[osl])
            pltpu.sync_copy(rows_b, ibr_out.at[osl])

    return gather_kernel(uidx_hi2, iidx_hi2, uidx_hi7, iidx_hi7,
                         uemb_r, iemb_r, ub_r, ib_r)


def _mlp_body(urows_ref, irows_ref, ubr_ref, ibr_ref, us_ref, is_ref,
              ulo_ref, ilo_ref, w1u_ref, w1i_ref, b1_ref, w2_ref, b2_ref,
              w3_ref, cst_ref, out_ref):
    d = w1u_ref.shape[0]
    bm = urows_ref.shape[0]
    us = us_ref[...]
    isx = is_ref[...]
    urows = urows_ref[...]
    irows = irows_ref[...]
    uvec = jnp.zeros((bm, d), jnp.float32)
    ivec = jnp.zeros((bm, d), jnp.float32)
    for k in range(4):
        um = (us == k).astype(jnp.float32)[:, None]
        im = (isx == k).astype(jnp.float32)[:, None]
        uvec = uvec + um * urows[:, k * d:(k + 1) * d]
        ivec = ivec + im * irows[:, k * d:(k + 1) * d]
    lane = jax.lax.broadcasted_iota(jnp.int32, (bm, 128), 1)
    um = (lane == ulo_ref[...][:, None]).astype(jnp.float32)
    im = (lane == ilo_ref[...][:, None]).astype(jnp.float32)
    ub = jnp.sum(ubr_ref[...] * um, axis=1)
    ib = jnp.sum(ibr_ref[...] * im, axis=1)
    h = (jnp.dot(uvec, w1u_ref[...], preferred_element_type=jnp.float32)
         + jnp.dot(ivec, w1i_ref[...], preferred_element_type=jnp.float32)
         + b1_ref[...])
    h = jnp.maximum(h, 0.0)
    h2 = jnp.dot(h, w2_ref[...], preferred_element_type=jnp.float32) + b2_ref[...]
    h2 = jnp.maximum(h2, 0.0)
    inter = jnp.sum(h2 * w3_ref[...], axis=1)
    pred = cst_ref[0, 0] + ub + ib + inter
    out_ref[...] = jnp.clip(pred, _MIN_R, _MAX_R)


@functools.partial(jax.jit, static_argnums=(15, 16, 17, 18))
def _tc_mlp(urows, irows, ubr, ibr, us, isx, ulo, ilo,
            w1u, w1i, b1r, w2t, b2r, w3r, cst, B, D, H, W):
    bm = 2048
    grid = (B // bm,)
    return pl.pallas_call(
        _mlp_body,
        grid=grid,
        in_specs=[
            pl.BlockSpec((bm, W), lambda i: (i, 0)),
            pl.BlockSpec((bm, W), lambda i: (i, 0)),
            pl.BlockSpec((bm, W), lambda i: (i, 0)),
            pl.BlockSpec((bm, W), lambda i: (i, 0)),
            pl.BlockSpec((bm,), lambda i: (i,)),
            pl.BlockSpec((bm,), lambda i: (i,)),
            pl.BlockSpec((bm,), lambda i: (i,)),
            pl.BlockSpec((bm,), lambda i: (i,)),
            pl.BlockSpec((D, H), lambda i: (0, 0)),
            pl.BlockSpec((D, H), lambda i: (0, 0)),
            pl.BlockSpec((1, H), lambda i: (0, 0)),
            pl.BlockSpec((H, 32), lambda i: (0, 0)),
            pl.BlockSpec((1, 32), lambda i: (0, 0)),
            pl.BlockSpec((1, 32), lambda i: (0, 0)),
            pl.BlockSpec((1, 1), lambda i: (0, 0)),
        ],
        out_specs=pl.BlockSpec((bm,), lambda i: (i,)),
        out_shape=jax.ShapeDtypeStruct((B,), jnp.float32),
    )(urows, irows, ubr, ibr, us, isx, ulo, ilo,
      w1u, w1i, b1r, w2t, b2r, w3r, cst)


def kernel(user_idx, item_idx, user_emb, item_emb, user_bias, item_bias,
           W1, b1, W2, b2, W3, b3):
    B = user_idx.shape[0]
    U, D = user_emb.shape
    H = W1.shape[0]
    W = 4 * D

    uemb_r = user_emb.reshape(U // 4, W)
    iemb_r = _tc_transpose(item_emb.T).reshape(U // 4, W)
    uidx_hi2 = jax.lax.shift_right_logical(user_idx, 2)
    iidx_hi2 = jax.lax.shift_right_logical(item_idx, 2)
    us = jax.lax.bitwise_and(user_idx, 3)
    isx = jax.lax.bitwise_and(item_idx, 3)
    uidx_hi7 = jax.lax.shift_right_logical(user_idx, 7)
    iidx_hi7 = jax.lax.shift_right_logical(item_idx, 7)
    uidx_lo7 = jax.lax.bitwise_and(user_idx, 127)
    iidx_lo7 = jax.lax.bitwise_and(item_idx, 127)

    u_pad = (-U) % 128
    ub_r = jnp.pad(user_bias.reshape(-1), (0, u_pad)).reshape(-1, 128)
    ib_r = jnp.pad(item_bias.reshape(-1), (0, u_pad)).reshape(-1, 128)

    urows, irows, ubr, ibr = _sc_gather(
        uidx_hi2, iidx_hi2, uidx_hi7, iidx_hi7,
        uemb_r, iemb_r, ub_r, ib_r, B, W)

    w1u = W1[:, :D].T
    w1i = W1[:, D:].T
    b1r = b1.reshape(1, H)
    w2t = W2.T
    b2r = b2.reshape(1, 32)
    w3r = W3.reshape(1, 32)
    cst = (_GLOBAL_MEAN + b3).reshape(1, 1)

    return _tc_mlp(urows, irows, ubr, ibr, us, isx, uidx_lo7, iidx_lo7,
                   w1u, w1i, b1r, w2t, b2r, w3r, cst, B, D, H, W)


# speedup vs baseline: 1.9797x; 1.7416x over previous
import functools

import jax
import jax.numpy as jnp
from jax import lax
from jax.experimental import pallas as pl
from jax.experimental.pallas import tpu as pltpu
from jax.experimental.pallas import tpu_sc as plsc

_GLOBAL_MEAN = 3.5
_MIN_R = 1.0
_MAX_R = 5.0


def _tp_body(in_ref, eye_ref, out_ref):
    d, bn = in_ref.shape
    t = jax.lax.dot_general(
        in_ref[...], eye_ref[...], (((0,), (0,)), ((), ())),
        preferred_element_type=jnp.float32)
    q = bn // 4
    out_ref[...] = jnp.concatenate(
        [t[k * q:(k + 1) * q, :] for k in range(4)], axis=1)


_BN = 16384


@jax.jit
def _tc_transpose(tab_t):
    d, u = tab_t.shape
    bn = _BN
    nblk = pl.cdiv(u, bn)
    return pl.pallas_call(
        _tp_body,
        grid=(nblk,),
        in_specs=[
            pl.BlockSpec((d, bn), lambda i: (0, i)),
            pl.BlockSpec((d, d), lambda i: (0, 0)),
        ],
        out_specs=pl.BlockSpec((bn // 4, 4 * d), lambda i: (i, 0)),
        out_shape=jax.ShapeDtypeStruct((nblk * bn // 4, 4 * d), jnp.float32),
    )(tab_t, jnp.eye(d, dtype=jnp.float32))


@functools.partial(jax.jit, static_argnums=(8, 9))
def _sc_gather(uidx_hi2, iidx_hi2, uidx_hi7, iidx_hi7,
               uemb_r, iemb_r, ub_r, ib_r, B, W):
    info = plsc.get_sparse_core_info()
    nw = info.num_cores * info.num_subcores
    nc = info.num_cores
    b_per_w = B // nw
    chunk = b_per_w // 2
    mesh = plsc.VectorSubcoreMesh(core_axis_name="c", subcore_axis_name="s")

    @functools.partial(
        pl.kernel,
        out_type=(
            jax.ShapeDtypeStruct((B, W), jnp.float32),
            jax.ShapeDtypeStruct((B, W), jnp.float32),
            jax.ShapeDtypeStruct((B, W), jnp.float32),
            jax.ShapeDtypeStruct((B, W), jnp.float32),
        ),
        mesh=mesh,
        scratch_types=[
            pltpu.VMEM((b_per_w,), jnp.int32),
            pltpu.VMEM((b_per_w,), jnp.int32),
            pltpu.VMEM((b_per_w,), jnp.int32),
            pltpu.VMEM((b_per_w,), jnp.int32),
            pltpu.VMEM((chunk, W), jnp.float32),
            pltpu.VMEM((chunk, W), jnp.float32),
            pltpu.SemaphoreType.DMA,
        ],
    )
    def gather_kernel(uhi2_hbm, ihi2_hbm, uhi7_hbm, ihi7_hbm,
                      uemb_hbm, iemb_hbm, ubr_hbm, ibr_hbm,
                      urows_out, irows_out, ubr_out, ibr_out,
                      uhi2_v, ihi2_v, uhi7_v, ihi7_v,
                      rows_a, rows_b, sem):
        wid = lax.axis_index("s") * nc + lax.axis_index("c")
        base = wid * b_per_w
        bsl = pl.ds(base, b_per_w)
        pltpu.sync_copy(uhi2_hbm.at[bsl], uhi2_v)
        pltpu.sync_copy(ihi2_hbm.at[bsl], ihi2_v)
        pltpu.sync_copy(uhi7_hbm.at[bsl], uhi7_v)
        pltpu.sync_copy(ihi7_hbm.at[bsl], ihi7_v)
        for half in range(2):
            lo = half * chunk
            csl = pl.ds(lo, chunk)
            osl = pl.ds(base + lo, chunk)
            cu = pltpu.async_copy(uemb_hbm.at[uhi2_v.at[csl]], rows_a, sem)
            ci = pltpu.async_copy(iemb_hbm.at[ihi2_v.at[csl]], rows_b, sem)
            cu.wait()
            ci.wait()
            pltpu.sync_copy(rows_a, urows_out.at[osl])
            pltpu.sync_copy(rows_b, irows_out.at[osl])
        for half in range(2):
            lo = half * chunk
            csl = pl.ds(lo, chunk)
            osl = pl.ds(base + lo, chunk)
            cu = pltpu.async_copy(ubr_hbm.at[uhi7_v.at[csl]], rows_a, sem)
            ci = pltpu.async_copy(ibr_hbm.at[ihi7_v.at[csl]], rows_b, sem)
            cu.wait()
            ci.wait()
            pltpu.sync_copy(rows_a, ubr_out.at[osl])
            pltpu.sync_copy(rows_b, ibr_out.at[osl])

    return gather_kernel(uidx_hi2, iidx_hi2, uidx_hi7, iidx_hi7,
                         uemb_r, iemb_r, ub_r, ib_r)


def _mlp_body(urows_ref, irows_ref, ubr_ref, ibr_ref, us_ref, is_ref,
              ulo_ref, ilo_ref, w1u_ref, w1i_ref, b1_ref, w2_ref, b2_ref,
              w3_ref, cst_ref, out_ref):
    d = w1u_ref.shape[0]
    bm = urows_ref.shape[0]
    us = us_ref[...]
    isx = is_ref[...]
    urows = urows_ref[...]
    irows = irows_ref[...]
    uvec = jnp.zeros((bm, d), jnp.float32)
    ivec = jnp.zeros((bm, d), jnp.float32)
    for k in range(4):
        um = (us == k).astype(jnp.float32)[:, None]
        im = (isx == k).astype(jnp.float32)[:, None]
        uvec = uvec + um * urows[:, k * d:(k + 1) * d]
        ivec = ivec + im * irows[:, k * d:(k + 1) * d]
    lane = jax.lax.broadcasted_iota(jnp.int32, (bm, 128), 1)
    um = (lane == ulo_ref[...][:, None]).astype(jnp.float32)
    im = (lane == ilo_ref[...][:, None]).astype(jnp.float32)
    ub = jnp.sum(ubr_ref[...] * um, axis=1)
    ib = jnp.sum(ibr_ref[...] * im, axis=1)
    h = (jnp.dot(uvec, w1u_ref[...], preferred_element_type=jnp.float32)
         + jnp.dot(ivec, w1i_ref[...], preferred_element_type=jnp.float32)
         + b1_ref[...])
    h = jnp.maximum(h, 0.0)
    h2 = jnp.dot(h, w2_ref[...], preferred_element_type=jnp.float32) + b2_ref[...]
    h2 = jnp.maximum(h2, 0.0)
    inter = jnp.sum(h2 * w3_ref[...], axis=1)
    pred = cst_ref[0, 0] + ub + ib + inter
    out_ref[...] = jnp.clip(pred, _MIN_R, _MAX_R)


@functools.partial(jax.jit, static_argnums=(15, 16, 17, 18))
def _tc_mlp(urows, irows, ubr, ibr, us, isx, ulo, ilo,
            w1u, w1i, b1r, w2t, b2r, w3r, cst, B, D, H, W):
    bm = 2048
    grid = (B // bm,)
    return pl.pallas_call(
        _mlp_body,
        grid=grid,
        in_specs=[
            pl.BlockSpec((bm, W), lambda i: (i, 0)),
            pl.BlockSpec((bm, W), lambda i: (i, 0)),
            pl.BlockSpec((bm, W), lambda i: (i, 0)),
            pl.BlockSpec((bm, W), lambda i: (i, 0)),
            pl.BlockSpec((bm,), lambda i: (i,)),
            pl.BlockSpec((bm,), lambda i: (i,)),
            pl.BlockSpec((bm,), lambda i: (i,)),
            pl.BlockSpec((bm,), lambda i: (i,)),
            pl.BlockSpec((D, H), lambda i: (0, 0)),
            pl.BlockSpec((D, H), lambda i: (0, 0)),
            pl.BlockSpec((1, H), lambda i: (0, 0)),
            pl.BlockSpec((H, 32), lambda i: (0, 0)),
            pl.BlockSpec((1, 32), lambda i: (0, 0)),
            pl.BlockSpec((1, 32), lambda i: (0, 0)),
            pl.BlockSpec((1, 1), lambda i: (0, 0)),
        ],
        out_specs=pl.BlockSpec((bm,), lambda i: (i,)),
        out_shape=jax.ShapeDtypeStruct((B,), jnp.float32),
    )(urows, irows, ubr, ibr, us, isx, ulo, ilo,
      w1u, w1i, b1r, w2t, b2r, w3r, cst)


def kernel(user_idx, item_idx, user_emb, item_emb, user_bias, item_bias,
           W1, b1, W2, b2, W3, b3):
    B = user_idx.shape[0]
    U, D = user_emb.shape
    H = W1.shape[0]
    W = 4 * D

    uemb_r = _tc_transpose(user_emb.T)
    iemb_r = _tc_transpose(item_emb.T)
    up = jax.lax.bitwise_and(user_idx, _BN - 1)
    uidx_hi2 = (jax.lax.shift_right_logical(user_idx, 14) * (_BN // 4)
                + jax.lax.bitwise_and(up, _BN // 4 - 1))
    us = jax.lax.shift_right_logical(up, 12)
    ip = jax.lax.bitwise_and(item_idx, _BN - 1)
    iidx_hi2 = (jax.lax.shift_right_logical(item_idx, 14) * (_BN // 4)
                + jax.lax.bitwise_and(ip, _BN // 4 - 1))
    isx = jax.lax.shift_right_logical(ip, 12)
    uidx_hi7 = jax.lax.shift_right_logical(user_idx, 7)
    iidx_hi7 = jax.lax.shift_right_logical(item_idx, 7)
    uidx_lo7 = jax.lax.bitwise_and(user_idx, 127)
    iidx_lo7 = jax.lax.bitwise_and(item_idx, 127)

    u_pad = (-U) % 128
    ub_r = jnp.pad(user_bias.reshape(-1), (0, u_pad)).reshape(-1, 128)
    ib_r = jnp.pad(item_bias.reshape(-1), (0, u_pad)).reshape(-1, 128)

    urows, irows, ubr, ibr = _sc_gather(
        uidx_hi2, iidx_hi2, uidx_hi7, iidx_hi7,
        uemb_r, iemb_r, ub_r, ib_r, B, W)

    w1u = W1[:, :D].T
    w1i = W1[:, D:].T
    b1r = b1.reshape(1, H)
    w2t = W2.T
    b2r = b2.reshape(1, 32)
    w3r = W3.reshape(1, 32)
    cst = (_GLOBAL_MEAN + b3).reshape(1, 1)

    return _tc_mlp(urows, irows, ubr, ibr, us, isx, uidx_lo7, iidx_lo7,
                   w1u, w1i, b1r, w2t, b2r, w3r, cst, B, D, H, W)
